# Initial kernel scaffold; baseline (speedup 1.0000x reference)
import jax
import jax.numpy as jnp
from jax.experimental import pallas as pl


def kernel(x, w1, b1, w2, b2, wf1_t, bf1_2d, wf2_t, bf2_2d):
    raise NotImplementedError("write your pallas kernel here")


# speedup vs baseline: 1.6270x; 1.6270x over previous
import jax
import jax.numpy as jnp
from jax.experimental import pallas as pl
from jax.experimental.pallas import tpu as pltpu


def _convs_kernel(w1_ref, b1_ref, w2_ref, b2_ref, x_ref, o_ref, p1_scr):
    for a in range(2):
        for b in range(2):
            accs = [jnp.full((63, 63), b1_ref[cl], dtype=jnp.float32)
                    for cl in range(6)]
            for ci in range(3):
                for kh in range(3):
                    pa, dr = (a + kh) % 2, (a + kh) // 2
                    for kw in range(3):
                        pb, dc = (b + kw) % 2, (b + kw) // 2
                        sl = x_ref[0, ci * 4 + pa * 2 + pb,
                                   dr * 9:dr * 9 + 63, dc * 9:dc * 9 + 63]
                        for cl in range(6):
                            w = w1_ref[((cl * 3 + ci) * 3 + kh) * 3 + kw]
                            accs[cl] = accs[cl] + w * sl
            ph = a * 2 + b
            for cl in range(6):
                if ph == 0:
                    p1_scr[cl] = accs[cl]
                elif ph == 3:
                    p1_scr[cl] = jnp.maximum(
                        jnp.maximum(p1_scr[cl], accs[cl]), 0.0)
                else:
                    p1_scr[cl] = jnp.maximum(p1_scr[cl], accs[cl])

    accs2 = [jnp.full((27, 27), b2_ref[co], dtype=jnp.float32)
             for co in range(16)]
    for ci in range(6):
        for kh in range(5):
            for kw in range(5):
                sl = p1_scr[ci, kh * 9:kh * 9 + 27, kw * 9:kw * 9 + 27]
                for co in range(16):
                    w = w2_ref[((co * 6 + ci) * 5 + kh) * 5 + kw]
                    accs2[co] = accs2[co] + w * sl
    for co in range(16):
        a2 = accs2[co]
        m = jnp.maximum(jnp.maximum(a2[0:9], a2[9:18]), a2[18:27])
        m = jnp.maximum(jnp.maximum(m[:, 0:9], m[:, 9:18]), m[:, 18:27])
        o_ref[0, co] = jnp.maximum(m, 0.0)


def _fc_kernel(x_ref, wa_ref, ba_ref, wb_ref, bb_ref, o_ref):
    h = jnp.dot(x_ref[...], wa_ref[...],
                preferred_element_type=jnp.float32) + ba_ref[...]
    o_ref[...] = jnp.dot(h, wb_ref[...],
                         preferred_element_type=jnp.float32) + bb_ref[...]


def kernel(x, w1, b1, w2, b2, wf1_t, bf1_2d, wf2_t, bf2_2d):
    n = x.shape[0]

    xp = jnp.pad(x[:, :, :215, :215], ((0, 0), (0, 0), (1, 0), (1, 0)))
    xr = xp.reshape(n, 3, 9, 24, 9, 24)[:, :, :, :16, :, :16]
    xr = xr.reshape(n, 3, 9, 8, 2, 9, 8, 2)
    xr = xr.transpose(0, 1, 4, 7, 3, 2, 6, 5).reshape(n, 12, 72, 72)

    p2 = pl.pallas_call(
        _convs_kernel,
        out_shape=jax.ShapeDtypeStruct((n, 16, 9, 9), jnp.float32),
        grid=(n,),
        in_specs=[
            pl.BlockSpec(memory_space=pltpu.MemorySpace.SMEM),
            pl.BlockSpec(memory_space=pltpu.MemorySpace.SMEM),
            pl.BlockSpec(memory_space=pltpu.MemorySpace.SMEM),
            pl.BlockSpec(memory_space=pltpu.MemorySpace.SMEM),
            pl.BlockSpec((1, 12, 72, 72), lambda i: (i, 0, 0, 0)),
        ],
        out_specs=pl.BlockSpec((1, 16, 9, 9), lambda i: (i, 0, 0, 0)),
        scratch_shapes=[pltpu.VMEM((6, 63, 63), jnp.float32)],
        compiler_params=pltpu.CompilerParams(
            dimension_semantics=("parallel",)),
    )(w1.reshape(-1), b1, w2.reshape(-1), b2, xr)

    flat = p2.reshape(n, 16 * 9 * 9)

    return pl.pallas_call(
        _fc_kernel,
        out_shape=jax.ShapeDtypeStruct((n, 10), jnp.float32),
        grid=(1,),
        in_specs=[
            pl.BlockSpec((n, 1296), lambda i: (0, 0)),
            pl.BlockSpec((1296, 360), lambda i: (0, 0)),
            pl.BlockSpec((1, 360), lambda i: (0, 0)),
            pl.BlockSpec((360, 10), lambda i: (0, 0)),
            pl.BlockSpec((1, 10), lambda i: (0, 0)),
        ],
        out_specs=pl.BlockSpec((n, 10), lambda i: (0, 0)),
        compiler_params=pltpu.CompilerParams(
            dimension_semantics=("arbitrary",)),
    )(flat, wf1_t, bf1_2d, wf2_t, bf2_2d)


# speedup vs baseline: 2.3637x; 1.4529x over previous
import jax
import jax.numpy as jnp
from jax.experimental import pallas as pl
from jax.experimental.pallas import tpu as pltpu


def _convs_kernel(w1_ref, b1_ref, w2_ref, b2_ref, x_ref, o_ref, p1_scr, col_scr):
    for a in range(2):
        for b in range(2):
            accs = [jnp.full((63, 63), b1_ref[cl], dtype=jnp.float32)
                    for cl in range(6)]
            for ci in range(3):
                for kh in range(3):
                    pa, dr = (a + kh) % 2, (a + kh) // 2
                    for kw in range(3):
                        pb, dc = (b + kw) % 2, (b + kw) // 2
                        sl = x_ref[0, ci * 4 + pa * 2 + pb,
                                   dr * 9:dr * 9 + 63, dc * 9:dc * 9 + 63]
                        for cl in range(6):
                            w = w1_ref[((cl * 3 + ci) * 3 + kh) * 3 + kw]
                            accs[cl] = accs[cl] + w * sl
            ph = a * 2 + b
            for cl in range(6):
                if ph == 0:
                    p1_scr[cl] = accs[cl]
                elif ph == 3:
                    p1_scr[cl] = jnp.maximum(
                        jnp.maximum(p1_scr[cl], accs[cl]), 0.0)
                else:
                    p1_scr[cl] = jnp.maximum(p1_scr[cl], accs[cl])

    for ci in range(6):
        for kw in range(5):
            col_scr[ci * 5 + kw, 0:63, :] = p1_scr[ci, :, kw * 9:kw * 9 + 27]
    for cg in range(4):
        accs2 = [jnp.full((27, 27), b2_ref[cg * 4 + cl], dtype=jnp.float32)
                 for cl in range(4)]
        for ci in range(6):
            for kh in range(5):
                for kw in range(5):
                    sl = col_scr[ci * 5 + kw, kh * 9:kh * 9 + 27, :]
                    for cl in range(4):
                        co = cg * 4 + cl
                        w = w2_ref[((co * 6 + ci) * 5 + kh) * 5 + kw]
                        accs2[cl] = accs2[cl] + w * sl
        for cl in range(4):
            a2 = accs2[cl]
            m = jnp.maximum(jnp.maximum(a2[0:9], a2[9:18]), a2[18:27])
            m = jnp.maximum(jnp.maximum(m[:, 0:9], m[:, 9:18]), m[:, 18:27])
            o_ref[0, cg * 4 + cl] = jnp.maximum(m, 0.0)


def _fc_kernel(x_ref, wa_ref, ba_ref, wb_ref, bb_ref, o_ref):
    h = jnp.dot(x_ref[...], wa_ref[...],
                preferred_element_type=jnp.float32) + ba_ref[...]
    o_ref[...] = jnp.dot(h, wb_ref[...],
                         preferred_element_type=jnp.float32) + bb_ref[...]


def kernel(x, w1, b1, w2, b2, wf1_t, bf1_2d, wf2_t, bf2_2d):
    n = x.shape[0]

    xp = jnp.pad(x[:, :, :215, :215], ((0, 0), (0, 0), (1, 0), (1, 0)))
    xr = xp.reshape(n, 3, 9, 24, 9, 24)[:, :, :, :16, :, :16]
    xr = xr.reshape(n, 3, 9, 8, 2, 9, 8, 2)
    xr = xr.transpose(0, 1, 4, 7, 3, 2, 6, 5).reshape(n, 12, 72, 72)

    p2 = pl.pallas_call(
        _convs_kernel,
        out_shape=jax.ShapeDtypeStruct((n, 16, 9, 9), jnp.float32),
        grid=(n,),
        in_specs=[
            pl.BlockSpec(memory_space=pltpu.MemorySpace.SMEM),
            pl.BlockSpec(memory_space=pltpu.MemorySpace.SMEM),
            pl.BlockSpec(memory_space=pltpu.MemorySpace.SMEM),
            pl.BlockSpec(memory_space=pltpu.MemorySpace.SMEM),
            pl.BlockSpec((1, 12, 72, 72), lambda i: (i, 0, 0, 0)),
        ],
        out_specs=pl.BlockSpec((1, 16, 9, 9), lambda i: (i, 0, 0, 0)),
        scratch_shapes=[pltpu.VMEM((6, 63, 63), jnp.float32),
                        pltpu.VMEM((30, 64, 27), jnp.float32)],
        compiler_params=pltpu.CompilerParams(
            dimension_semantics=("parallel",)),
    )(w1.reshape(-1), b1, w2.reshape(-1), b2, xr)

    flat = p2.reshape(n, 16 * 9 * 9)

    return pl.pallas_call(
        _fc_kernel,
        out_shape=jax.ShapeDtypeStruct((n, 10), jnp.float32),
        grid=(1,),
        in_specs=[
            pl.BlockSpec((n, 1296), lambda i: (0, 0)),
            pl.BlockSpec((1296, 360), lambda i: (0, 0)),
            pl.BlockSpec((1, 360), lambda i: (0, 0)),
            pl.BlockSpec((360, 10), lambda i: (0, 0)),
            pl.BlockSpec((1, 10), lambda i: (0, 0)),
        ],
        out_specs=pl.BlockSpec((n, 10), lambda i: (0, 0)),
        compiler_params=pltpu.CompilerParams(
            dimension_semantics=("arbitrary",)),
    )(flat, wf1_t, bf1_2d, wf2_t, bf2_2d)


# speedup vs baseline: 2.3850x; 1.0090x over previous
import jax
import jax.numpy as jnp
from jax.experimental import pallas as pl
from jax.experimental.pallas import tpu as pltpu


def _convs_kernel(w1_ref, b1_ref, w2_ref, b2_ref, x_ref, o_ref, p1_scr, col_scr,
                  c2_scr):
    for a in range(2):
        for b in range(2):
            ph = a * 2 + b
            for rt, rn in ((0, 16), (16, 16), (32, 16), (48, 15)):
                accs = [jnp.full((rn, 63), b1_ref[cl], dtype=jnp.float32)
                        for cl in range(6)]
                for ci in range(3):
                    for kh in range(3):
                        pa, dr = (a + kh) % 2, (a + kh) // 2
                        for kw in range(3):
                            pb, dc = (b + kw) % 2, (b + kw) // 2
                            sl = x_ref[0, ci * 4 + pa * 2 + pb,
                                       dr * 9 + rt:dr * 9 + rt + rn,
                                       dc * 9:dc * 9 + 63]
                            for cl in range(6):
                                w = w1_ref[((cl * 3 + ci) * 3 + kh) * 3 + kw]
                                accs[cl] = accs[cl] + w * sl
                for cl in range(6):
                    if ph == 0:
                        p1_scr[cl, rt:rt + rn] = accs[cl]
                    elif ph == 3:
                        p1_scr[cl, rt:rt + rn] = jnp.maximum(
                            jnp.maximum(p1_scr[cl, rt:rt + rn], accs[cl]), 0.0)
                    else:
                        p1_scr[cl, rt:rt + rn] = jnp.maximum(
                            p1_scr[cl, rt:rt + rn], accs[cl])

    for ci in range(6):
        for kw in range(5):
            col_scr[ci * 5 + kw, 0:63, :] = p1_scr[ci, :, kw * 9:kw * 9 + 27]
    for rt, rn in ((0, 16), (16, 11)):
        accs2 = [jnp.full((rn, 27), b2_ref[co], dtype=jnp.float32)
                 for co in range(16)]
        for ci in range(6):
            for kh in range(5):
                for kw in range(5):
                    sl = col_scr[ci * 5 + kw, kh * 9 + rt:kh * 9 + rt + rn, :]
                    for co in range(16):
                        w = w2_ref[((co * 6 + ci) * 5 + kh) * 5 + kw]
                        accs2[co] = accs2[co] + w * sl
        for co in range(16):
            c2_scr[co, rt:rt + rn] = accs2[co]
    for co in range(16):
        a2 = c2_scr[co]
        m = jnp.maximum(jnp.maximum(a2[0:9], a2[9:18]), a2[18:27])
        m = jnp.maximum(jnp.maximum(m[:, 0:9], m[:, 9:18]), m[:, 18:27])
        o_ref[0, co] = jnp.maximum(m, 0.0)


def _fc_kernel(x_ref, wa_ref, ba_ref, wb_ref, bb_ref, o_ref):
    h = jnp.dot(x_ref[...], wa_ref[...],
                preferred_element_type=jnp.float32) + ba_ref[...]
    o_ref[...] = jnp.dot(h, wb_ref[...],
                         preferred_element_type=jnp.float32) + bb_ref[...]


def kernel(x, w1, b1, w2, b2, wf1_t, bf1_2d, wf2_t, bf2_2d):
    n = x.shape[0]

    xp = jnp.pad(x[:, :, :215, :215], ((0, 0), (0, 0), (1, 0), (1, 0)))
    xr = xp.reshape(n, 3, 9, 24, 9, 24)[:, :, :, :16, :, :16]
    xr = xr.reshape(n, 3, 9, 8, 2, 9, 8, 2)
    xr = xr.transpose(0, 1, 4, 7, 3, 2, 6, 5).reshape(n, 12, 72, 72)

    p2 = pl.pallas_call(
        _convs_kernel,
        out_shape=jax.ShapeDtypeStruct((n, 16, 9, 9), jnp.float32),
        grid=(n,),
        in_specs=[
            pl.BlockSpec(memory_space=pltpu.MemorySpace.SMEM),
            pl.BlockSpec(memory_space=pltpu.MemorySpace.SMEM),
            pl.BlockSpec(memory_space=pltpu.MemorySpace.SMEM),
            pl.BlockSpec(memory_space=pltpu.MemorySpace.SMEM),
            pl.BlockSpec((1, 12, 72, 72), lambda i: (i, 0, 0, 0)),
        ],
        out_specs=pl.BlockSpec((1, 16, 9, 9), lambda i: (i, 0, 0, 0)),
        scratch_shapes=[pltpu.VMEM((6, 63, 63), jnp.float32),
                        pltpu.VMEM((30, 64, 27), jnp.float32),
                        pltpu.VMEM((16, 32, 27), jnp.float32)],
        compiler_params=pltpu.CompilerParams(
            dimension_semantics=("arbitrary",)),
    )(w1.reshape(-1), b1, w2.reshape(-1), b2, xr)

    flat = p2.reshape(n, 16 * 9 * 9)

    return pl.pallas_call(
        _fc_kernel,
        out_shape=jax.ShapeDtypeStruct((n, 10), jnp.float32),
        grid=(1,),
        in_specs=[
            pl.BlockSpec((n, 1296), lambda i: (0, 0)),
            pl.BlockSpec((1296, 360), lambda i: (0, 0)),
            pl.BlockSpec((1, 360), lambda i: (0, 0)),
            pl.BlockSpec((360, 10), lambda i: (0, 0)),
            pl.BlockSpec((1, 10), lambda i: (0, 0)),
        ],
        out_specs=pl.BlockSpec((n, 10), lambda i: (0, 0)),
        compiler_params=pltpu.CompilerParams(
            dimension_semantics=("arbitrary",)),
    )(flat, wf1_t, bf1_2d, wf2_t, bf2_2d)


# speedup vs baseline: 2.5314x; 1.0614x over previous
import numpy as np
import jax
import jax.numpy as jnp
from jax.experimental import pallas as pl
from jax.experimental.pallas import tpu as pltpu


def _convs_kernel(w1_ref, b1_ref, w2_ref, b2_ref, x_ref, r_ref, c_ref, o_ref,
                  p1_scr, col_scr, c2_scr, xph_scr):
    for ci in range(3):
        for pb in range(2):
            xph_scr[ci, pb] = jnp.dot(
                r_ref[...],
                jnp.dot(x_ref[0, ci], c_ref[pb],
                        preferred_element_type=jnp.float32),
                preferred_element_type=jnp.float32)

    for a in range(2):
        for b in range(2):
            ph = a * 2 + b
            for rt, rn in ((0, 16), (16, 16), (32, 16), (48, 15)):
                accs = [jnp.full((rn, 63), b1_ref[cl], dtype=jnp.float32)
                        for cl in range(6)]
                for ci in range(3):
                    for kh in range(3):
                        pa, dr = (a + kh) % 2, (a + kh) // 2
                        for kw in range(3):
                            pb, dc = (b + kw) % 2, (b + kw) // 2
                            r0 = pa * 72 + dr * 9 + rt
                            c0 = dc * 9
                            sl = xph_scr[ci, pb, r0:r0 + rn, c0:c0 + 63]
                            for cl in range(6):
                                w = w1_ref[((cl * 3 + ci) * 3 + kh) * 3 + kw]
                                accs[cl] = accs[cl] + w * sl
                for cl in range(6):
                    if ph == 0:
                        p1_scr[cl, rt:rt + rn] = accs[cl]
                    elif ph == 3:
                        p1_scr[cl, rt:rt + rn] = jnp.maximum(
                            jnp.maximum(p1_scr[cl, rt:rt + rn], accs[cl]), 0.0)
                    else:
                        p1_scr[cl, rt:rt + rn] = jnp.maximum(
                            p1_scr[cl, rt:rt + rn], accs[cl])

    for ci in range(6):
        for kw in range(5):
            col_scr[ci * 5 + kw, 0:63, :] = p1_scr[ci, :, kw * 9:kw * 9 + 27]
    for rt, rn in ((0, 16), (16, 11)):
        accs2 = [jnp.full((rn, 27), b2_ref[co], dtype=jnp.float32)
                 for co in range(16)]
        for ci in range(6):
            for kh in range(5):
                for kw in range(5):
                    sl = col_scr[ci * 5 + kw, kh * 9 + rt:kh * 9 + rt + rn, :]
                    for co in range(16):
                        w = w2_ref[((co * 6 + ci) * 5 + kh) * 5 + kw]
                        accs2[co] = accs2[co] + w * sl
        for co in range(16):
            c2_scr[co, rt:rt + rn] = accs2[co]
    for co in range(16):
        a2 = c2_scr[co]
        m = jnp.maximum(jnp.maximum(a2[0:9], a2[9:18]), a2[18:27])
        m = jnp.maximum(jnp.maximum(m[:, 0:9], m[:, 9:18]), m[:, 18:27])
        o_ref[0, co] = jnp.maximum(m, 0.0)


def _fc_kernel(x_ref, wa_ref, ba_ref, wb_ref, bb_ref, o_ref):
    h = jnp.dot(x_ref[...], wa_ref[...],
                preferred_element_type=jnp.float32) + ba_ref[...]
    o_ref[...] = jnp.dot(h, wb_ref[...],
                         preferred_element_type=jnp.float32) + bb_ref[...]


def kernel(x, w1, b1, w2, b2, wf1_t, bf1_2d, wf2_t, bf2_2d):
    n = x.shape[0]

    xp = jnp.pad(x[:, :, :215, :215], ((0, 0), (0, 0), (1, 0), (1, 0)))
    xs = xp.reshape(n, 3, 9, 24, 9, 24)[:, :, :, :17, :, :17]
    xs = xs.reshape(n, 3, 153, 153)

    r_np = np.zeros((144, 153), dtype=np.float32)
    c_np = np.zeros((2, 153, 72), dtype=np.float32)
    for pa in range(2):
        for v in range(8):
            for i in range(9):
                r_np[pa * 72 + v * 9 + i, i * 17 + 2 * v + pa] = 1.0
                c_np[pa, i * 17 + 2 * v + pa, v * 9 + i] = 1.0
    r_sel = jnp.asarray(r_np)
    c_sel = jnp.asarray(c_np)

    p2 = pl.pallas_call(
        _convs_kernel,
        out_shape=jax.ShapeDtypeStruct((n, 16, 9, 9), jnp.float32),
        grid=(n,),
        in_specs=[
            pl.BlockSpec(memory_space=pltpu.MemorySpace.SMEM),
            pl.BlockSpec(memory_space=pltpu.MemorySpace.SMEM),
            pl.BlockSpec(memory_space=pltpu.MemorySpace.SMEM),
            pl.BlockSpec(memory_space=pltpu.MemorySpace.SMEM),
            pl.BlockSpec((1, 3, 153, 153), lambda i: (i, 0, 0, 0)),
            pl.BlockSpec((144, 153), lambda i: (0, 0)),
            pl.BlockSpec((2, 153, 72), lambda i: (0, 0, 0)),
        ],
        out_specs=pl.BlockSpec((1, 16, 9, 9), lambda i: (i, 0, 0, 0)),
        scratch_shapes=[pltpu.VMEM((6, 63, 63), jnp.float32),
                        pltpu.VMEM((30, 64, 27), jnp.float32),
                        pltpu.VMEM((16, 32, 27), jnp.float32),
                        pltpu.VMEM((3, 2, 144, 72), jnp.float32)],
        compiler_params=pltpu.CompilerParams(
            dimension_semantics=("arbitrary",)),
    )(w1.reshape(-1), b1, w2.reshape(-1), b2, xs, r_sel, c_sel)

    flat = p2.reshape(n, 16 * 9 * 9)

    return pl.pallas_call(
        _fc_kernel,
        out_shape=jax.ShapeDtypeStruct((n, 10), jnp.float32),
        grid=(1,),
        in_specs=[
            pl.BlockSpec((n, 1296), lambda i: (0, 0)),
            pl.BlockSpec((1296, 360), lambda i: (0, 0)),
            pl.BlockSpec((1, 360), lambda i: (0, 0)),
            pl.BlockSpec((360, 10), lambda i: (0, 0)),
            pl.BlockSpec((1, 10), lambda i: (0, 0)),
        ],
        out_specs=pl.BlockSpec((n, 10), lambda i: (0, 0)),
        compiler_params=pltpu.CompilerParams(
            dimension_semantics=("arbitrary",)),
    )(flat, wf1_t, bf1_2d, wf2_t, bf2_2d)


# speedup vs baseline: 3.7544x; 1.4831x over previous
import numpy as np
import jax
import jax.numpy as jnp
from jax.experimental import pallas as pl
from jax.experimental.pallas import tpu as pltpu


def _convs_kernel(w1_ref, b1_ref, w2_ref, b2_ref, x_ref, r_ref, c_ref, o_ref,
                  p1_scr, col_scr, c2_scr, xph_scr):
    for ci in range(3):
        for pb in range(2):
            xph_scr[ci, pb] = jnp.dot(
                r_ref[...],
                jnp.dot(x_ref[0, ci], c_ref[pb],
                        preferred_element_type=jnp.float32),
                preferred_element_type=jnp.float32)

    for a in range(2):
        for b in range(2):
            ph = a * 2 + b
            for rt, rn in ((0, 16), (16, 16), (32, 16), (48, 15)):
                accs = [jnp.full((rn, 63), b1_ref[cl], dtype=jnp.float32)
                        for cl in range(6)]
                for ci in range(3):
                    for kh in range(3):
                        pa, dr = (a + kh) % 2, (a + kh) // 2
                        for kw in range(3):
                            pb, dc = (b + kw) % 2, (b + kw) // 2
                            r0 = pa * 72 + dr * 9 + rt
                            c0 = dc * 9
                            sl = xph_scr[ci, pb, r0:r0 + rn, c0:c0 + 63]
                            for cl in range(6):
                                w = w1_ref[((cl * 3 + ci) * 3 + kh) * 3 + kw]
                                accs[cl] = accs[cl] + w * sl
                for cl in range(6):
                    if ph == 0:
                        p1_scr[cl, rt:rt + rn] = accs[cl]
                    elif ph == 3:
                        p1_scr[cl, rt:rt + rn] = jnp.maximum(
                            jnp.maximum(p1_scr[cl, rt:rt + rn], accs[cl]), 0.0)
                    else:
                        p1_scr[cl, rt:rt + rn] = jnp.maximum(
                            p1_scr[cl, rt:rt + rn], accs[cl])

    for ci in range(6):
        for kw in range(5):
            col_scr[ci * 5 + kw, 0:63, :] = p1_scr[ci, :, kw * 9:kw * 9 + 27]
    for rt, rn in ((0, 16), (16, 11)):
        accs2 = [jnp.full((rn, 27), b2_ref[co], dtype=jnp.float32)
                 for co in range(16)]
        for ci in range(6):
            for kh in range(5):
                for kw in range(5):
                    sl = col_scr[ci * 5 + kw, kh * 9 + rt:kh * 9 + rt + rn, :]
                    for co in range(16):
                        w = w2_ref[((co * 6 + ci) * 5 + kh) * 5 + kw]
                        accs2[co] = accs2[co] + w * sl
        for co in range(16):
            c2_scr[co, rt:rt + rn] = accs2[co]
    for co in range(16):
        a2 = c2_scr[co]
        m = jnp.maximum(jnp.maximum(a2[0:9], a2[9:18]), a2[18:27])
        m = jnp.maximum(jnp.maximum(m[:, 0:9], m[:, 9:18]), m[:, 18:27])
        o_ref[0, co] = jnp.maximum(m, 0.0)


def _fc_kernel(x_ref, wa_ref, ba_ref, wb_ref, bb_ref, o_ref):
    h = jnp.dot(x_ref[...], wa_ref[...],
                preferred_element_type=jnp.float32) + ba_ref[...]
    o_ref[...] = jnp.dot(h, wb_ref[...],
                         preferred_element_type=jnp.float32) + bb_ref[...]


def kernel(x, w1, b1, w2, b2, wf1_t, bf1_2d, wf2_t, bf2_2d):
    n = x.shape[0]

    r_np = np.zeros((144, 224), dtype=np.float32)
    c_np = np.zeros((2, 224, 72), dtype=np.float32)
    for pa in range(2):
        for v in range(8):
            for i in range(9):
                src = 24 * i + 2 * v + pa - 1
                if src >= 0:
                    r_np[pa * 72 + v * 9 + i, src] = 1.0
                    c_np[pa, src, v * 9 + i] = 1.0
    r_sel = jnp.asarray(r_np)
    c_sel = jnp.asarray(c_np)

    p2 = pl.pallas_call(
        _convs_kernel,
        out_shape=jax.ShapeDtypeStruct((n, 16, 9, 9), jnp.float32),
        grid=(n,),
        in_specs=[
            pl.BlockSpec(memory_space=pltpu.MemorySpace.SMEM),
            pl.BlockSpec(memory_space=pltpu.MemorySpace.SMEM),
            pl.BlockSpec(memory_space=pltpu.MemorySpace.SMEM),
            pl.BlockSpec(memory_space=pltpu.MemorySpace.SMEM),
            pl.BlockSpec((1, 3, 224, 224), lambda i: (i, 0, 0, 0)),
            pl.BlockSpec((144, 224), lambda i: (0, 0)),
            pl.BlockSpec((2, 224, 72), lambda i: (0, 0, 0)),
        ],
        out_specs=pl.BlockSpec((1, 16, 9, 9), lambda i: (i, 0, 0, 0)),
        scratch_shapes=[pltpu.VMEM((6, 63, 63), jnp.float32),
                        pltpu.VMEM((30, 64, 27), jnp.float32),
                        pltpu.VMEM((16, 32, 27), jnp.float32),
                        pltpu.VMEM((3, 2, 144, 72), jnp.float32)],
        compiler_params=pltpu.CompilerParams(
            dimension_semantics=("arbitrary",)),
    )(w1.reshape(-1), b1, w2.reshape(-1), b2, x, r_sel, c_sel)

    flat = p2.reshape(n, 16 * 9 * 9)

    return pl.pallas_call(
        _fc_kernel,
        out_shape=jax.ShapeDtypeStruct((n, 10), jnp.float32),
        grid=(1,),
        in_specs=[
            pl.BlockSpec((n, 1296), lambda i: (0, 0)),
            pl.BlockSpec((1296, 360), lambda i: (0, 0)),
            pl.BlockSpec((1, 360), lambda i: (0, 0)),
            pl.BlockSpec((360, 10), lambda i: (0, 0)),
            pl.BlockSpec((1, 10), lambda i: (0, 0)),
        ],
        out_specs=pl.BlockSpec((n, 10), lambda i: (0, 0)),
        compiler_params=pltpu.CompilerParams(
            dimension_semantics=("arbitrary",)),
    )(flat, wf1_t, bf1_2d, wf2_t, bf2_2d)
